# baseline (speedup 1.0000x reference)
import functools
import numpy as np
import jax
import jax.numpy as jnp
from jax import lax
from jax.experimental import pallas as pl
from jax.experimental.pallas import tpu as pltpu
from jax.experimental.pallas import tpu_sc as plsc

NUMI = 16
PP = 16384
TT = 32
CC = 80

BPA = 16384
BCB = 8192
HALF = PP // 2

_BAL_B = float(np.e ** (1.5 / 0.5) - 1.0)


def _match_kernel(pr_ref, tr_ref, btv_ref, bti_ref, bpi_ref):
    tr = tr_ref[0]
    tx1 = tr[:, 0:1]
    ty1 = tr[:, 1:2]
    tx2 = tr[:, 2:3]
    ty2 = tr[:, 3:4]
    pr = pr_ref[...]
    cx = pr[0:1]
    cy = pr[1:2]
    w = pr[2:3]
    h = pr[3:4]
    px1 = cx - w / 2.0
    py1 = cy - h / 2.0
    px2 = cx + w / 2.0
    py2 = cy + h / 2.0
    iw = jnp.maximum(jnp.minimum(tx2, px2) - jnp.maximum(tx1, px1), 0.0)
    ih = jnp.maximum(jnp.minimum(ty2, py2) - jnp.maximum(ty1, py1), 0.0)
    inter = iw * ih
    area_t = (tx2 - tx1) * (ty2 - ty1)
    area_p = (px2 - px1) * (py2 - py1)
    ov = inter / (area_t + area_p - inter)

    btv = jnp.max(ov, axis=0, keepdims=True)
    ti = lax.broadcasted_iota(jnp.int32, ov.shape, 0)
    bti = jnp.min(jnp.where(ov == btv, ti, TT), axis=0, keepdims=True)
    btv_ref[0] = btv
    bti_ref[0] = bti

    rmax = jnp.max(ov, axis=1, keepdims=True)
    pi = lax.broadcasted_iota(jnp.int32, ov.shape, 1)
    rarg = jnp.min(jnp.where(ov == rmax, pi, PP), axis=1, keepdims=True)

    rarg_f = rarg.astype(jnp.float32)
    rarg_row = rarg_f.T
    t_c = lax.broadcasted_iota(jnp.int32, (TT, TT), 0)
    t_l = lax.broadcasted_iota(jnp.int32, (TT, TT), 1)
    eqm = jnp.where((rarg_row == rarg_f) & (t_l > t_c), 1, 0)
    dup = jnp.max(eqm, axis=1, keepdims=True) > 0
    tcol = lax.broadcasted_iota(jnp.int32, (TT, 1), 0)
    bpi_ref[0] = jnp.where(dup, PP + tcol, rarg)


def _sc_match(bti_hbm, btv_hbm, bpi_hbm, tt_hbm, btv2_hbm, m5_hbm,
              bti_v, btv_v, bpi_v, tr0, tr1, tr2, tr3, tr4,
              o0, o1, o2, o3, o4):
    wid = lax.axis_index("s") * 2 + lax.axis_index("c")
    n = wid // 2
    half = wid % 2
    base = half * HALF
    trs = (tr0, tr1, tr2, tr3, tr4)
    outs = (o0, o1, o2, o3, o4)
    pltpu.sync_copy(bti_hbm.at[n, 0, pl.ds(base, HALF)], bti_v)
    pltpu.sync_copy(btv_hbm.at[n, 0, pl.ds(base, HALF)], btv_v)
    pltpu.sync_copy(bpi_hbm.at[n, 0], bpi_v)
    for c in range(5):
        pltpu.sync_copy(tt_hbm.at[n, c, 0], trs[c])

    for hh in range(2):
        bp = bpi_v[pl.ds(hh * 16, 16)]
        il = bp - base
        mask = (il >= 0) & (il < HALF)
        ilc = jnp.clip(il, 0, HALF - 1)
        tvals = lax.iota(jnp.int32, 16) + (hh * 16)
        plsc.store_scatter(bti_v, [ilc], tvals, mask=mask)
        plsc.store_scatter(btv_v, [ilc], jnp.full((16,), 2.0, jnp.float32),
                           mask=mask)
    pltpu.sync_copy(btv_v, btv2_hbm.at[n, 0, pl.ds(base, HALF)])

    def body(i, carry):
        off = pl.multiple_of(i * 16, 16)
        idx = bti_v[pl.ds(off, 16)]
        for c in range(5):
            outs[c][pl.ds(off, 16)] = plsc.load_gather(trs[c], [idx])
        return carry

    lax.fori_loop(0, HALF // 16, body, 0)
    for c in range(5):
        pltpu.sync_copy(outs[c], m5_hbm.at[n, c, 0, pl.ds(base, HALF)])


def _loss_kernel(conf_ref, loct_ref, pr_ref, btv_ref, m5_ref,
                 out_l, out_c, out_n):
    n = pl.program_id(0)
    j = pl.program_id(1)
    first = jnp.logical_and(n == 0, j == 0)

    btv = btv_ref[0]
    m5 = m5_ref[0]

    pos = btv >= 0.5
    neg = btv < 0.4
    posf = pos.astype(jnp.float32)
    pnf = (pos | neg).astype(jnp.float32)

    pr = pr_ref[...]
    cx = pr[0:1]
    cy = pr[1:2]
    w = pr[2:3]
    h = pr[3:4]
    mx1 = m5[0:1]
    my1 = m5[1:2]
    mx2 = m5[2:3]
    my2 = m5[3:4]
    gcx = ((mx1 + mx2) / 2.0 - cx) / (0.1 * w)
    gcy = ((my1 + my2) / 2.0 - cy) / (0.1 * h)
    gw = jnp.log((mx2 - mx1) / w) / 0.2
    gh = jnp.log((my2 - my1) / h) / 0.2
    enc = jnp.concatenate([gcx, gcy, gw, gh], axis=0)

    diff = jnp.abs(loct_ref[0] - enc)
    ll = jnp.where(
        diff < 0.11,
        0.5 / _BAL_B * (_BAL_B * diff + 1.0) * jnp.log(_BAL_B * diff / 0.11 + 1.0)
        - 0.5 * diff,
        1.5 * diff + 1.5 / _BAL_B - 0.5 * 0.11)
    ll_sum = jnp.sum(ll * posf)

    kpos = jnp.where(pos, m5[4:5], -1.0)
    stacked = jnp.concatenate(
        [pnf, kpos, jnp.zeros((6, BCB), jnp.float32)], axis=0)
    cols = stacked.T
    pnc = cols[:, 0:1]
    kc = cols[:, 1:2]

    x = conf_ref[0]
    e = jnp.exp(-jnp.abs(x))
    u = 1.0 + e
    l1pe = jnp.log(u)
    ce0 = jnp.maximum(x, 0.0) + l1pe
    r = 1.0 / u
    er = e * r
    p = jnp.where(x >= 0, r, er)
    a75 = ce0 * (p * 0.75)
    b25 = (ce0 - x) * (0.25 - 0.25 * p)
    ciota = lax.broadcasted_iota(
        jnp.int32, (BCB, CC), 1).astype(jnp.float32)
    eqf = jnp.where(ciota == kc, 1.0, 0.0)
    contrib = a75 * (pnc - eqf) + b25 * eqf
    c_sum = jnp.sum(contrib)
    n_sum = jnp.sum(posf)

    @pl.when(first)
    def _():
        out_l[...] = jnp.zeros((1, 1), jnp.float32)
        out_c[...] = jnp.zeros((1, 1), jnp.float32)
        out_n[...] = jnp.zeros((1, 1), jnp.float32)

    out_l[...] += ll_sum.reshape(1, 1)
    out_c[...] += c_sum.reshape(1, 1)
    out_n[...] += n_sum.reshape(1, 1)


def _run(loc_data, conf_data, priors, targets, interpret=False):
    priors_t = priors.T
    loc_tr = jnp.transpose(loc_data, (0, 2, 1))
    btv, bti, bpi = pl.pallas_call(
        _match_kernel,
        grid=(NUMI, PP // BPA),
        in_specs=[
            pl.BlockSpec((4, BPA), lambda n, j: (0, j)),
            pl.BlockSpec((1, TT, 5), lambda n, j: (n, 0, 0)),
        ],
        out_specs=[
            pl.BlockSpec((1, 1, BPA), lambda n, j: (n, 0, j)),
            pl.BlockSpec((1, 1, BPA), lambda n, j: (n, 0, j)),
            pl.BlockSpec((1, TT, 1), lambda n, j: (n, 0, 0)),
        ],
        out_shape=[
            jax.ShapeDtypeStruct((NUMI, 1, PP), jnp.float32),
            jax.ShapeDtypeStruct((NUMI, 1, PP), jnp.int32),
            jax.ShapeDtypeStruct((NUMI, TT, 1), jnp.int32),
        ],
        interpret=interpret,
    )(priors_t, targets)

    tt = jnp.transpose(targets, (0, 2, 1)).reshape(NUMI, 5, 1, TT)
    sc_call = functools.partial(
        pl.kernel,
        out_type=[
            jax.ShapeDtypeStruct((NUMI, 1, PP), jnp.float32),
            jax.ShapeDtypeStruct((NUMI, 5, 1, PP), jnp.float32),
        ],
        mesh=plsc.VectorSubcoreMesh(core_axis_name="c", subcore_axis_name="s"),
        compiler_params=pltpu.CompilerParams(needs_layout_passes=False),
        scratch_types=[
            pltpu.VMEM((HALF,), jnp.int32),
            pltpu.VMEM((HALF,), jnp.float32),
            pltpu.VMEM((TT,), jnp.int32),
            pltpu.VMEM((TT,), jnp.float32),
            pltpu.VMEM((TT,), jnp.float32),
            pltpu.VMEM((TT,), jnp.float32),
            pltpu.VMEM((TT,), jnp.float32),
            pltpu.VMEM((TT,), jnp.float32),
            pltpu.VMEM((HALF,), jnp.float32),
            pltpu.VMEM((HALF,), jnp.float32),
            pltpu.VMEM((HALF,), jnp.float32),
            pltpu.VMEM((HALF,), jnp.float32),
            pltpu.VMEM((HALF,), jnp.float32),
        ],
    )(_sc_match)
    btv2, m5 = sc_call(bti, btv, bpi.reshape(NUMI, 1, TT), tt)
    m5 = m5.reshape(NUMI, 5, PP)

    sl, sc, sn = pl.pallas_call(
        _loss_kernel,
        grid=(NUMI, PP // BCB),
        in_specs=[
            pl.BlockSpec((1, BCB, CC), lambda n, j: (n, j, 0)),
            pl.BlockSpec((1, 4, BCB), lambda n, j: (n, 0, j)),
            pl.BlockSpec((4, BCB), lambda n, j: (0, j)),
            pl.BlockSpec((1, 1, BCB), lambda n, j: (n, 0, j)),
            pl.BlockSpec((1, 5, BCB), lambda n, j: (n, 0, j)),
        ],
        out_specs=[
            pl.BlockSpec((1, 1), lambda n, j: (0, 0)),
            pl.BlockSpec((1, 1), lambda n, j: (0, 0)),
            pl.BlockSpec((1, 1), lambda n, j: (0, 0)),
        ],
        out_shape=[
            jax.ShapeDtypeStruct((1, 1), jnp.float32),
            jax.ShapeDtypeStruct((1, 1), jnp.float32),
            jax.ShapeDtypeStruct((1, 1), jnp.float32),
        ],
        interpret=interpret,
    )(conf_data, loc_tr, priors_t, btv2, m5)

    pos_num = jnp.maximum(sn[0, 0], 1.0)
    loss_l = sl[0, 0] / (pos_num * 4.0)
    loss_c = sc[0, 0] / pos_num
    return (loss_l, loss_c)


@jax.jit
def kernel(loc_data, conf_data, priors, targets):
    return _run(loc_data, conf_data, priors, targets)
